# baseline (speedup 1.0000x reference)
import functools

import jax
import jax.numpy as jnp
from jax import lax
from jax.experimental import pallas as pl
from jax.experimental.pallas import tpu as pltpu
from jax.experimental.pallas import tpu_sc as plsc

_B, _N, _OV, _O = 16, 4096, 128, 64
_NW = 32
_PARTS = 4
_BPP = _B // _PARTS
_WPB = _NW // _BPP
_RW = _N // _WPB
_C = 128
_NC = _RW // _C
_NV = _O // 16


def _gs_body(global_state_ref, W_ref, gs_ref):
    gs_ref[...] = jnp.dot(global_state_ref[...], W_ref[...],
                          preferred_element_type=jnp.float32)


def _fill_tail(buf, vecs):
    def body(i, _):
        for j in range(_NV):
            buf[i, pl.ds(_OV + j * 16, 16)] = vecs[j]
        return 0
    lax.fori_loop(0, _C, body, 0)


def _make_sc_body(part):
    def _sc_body(V_hbm, gs_hbm, gsz_hbm, out_hbm,
                 gsz_v, gs_row_v, buf0, buf1, in_s0, in_s1, out_s0, out_s1):
        cid = lax.axis_index("c")
        sid = lax.axis_index("s")
        wid = sid * 2 + cid
        lb = wid // _WPB
        bidx = part * _BPP + lb
        r0 = (wid % _WPB) * _RW

        pltpu.sync_copy(gsz_hbm, gsz_v)
        pltpu.sync_copy(gs_hbm.at[bidx], gs_row_v)
        gvec = gsz_v[...]
        gsize = gvec[part * _BPP]
        for k in range(part * _BPP + 1, (part + 1) * _BPP):
            gsize = jnp.where(bidx == k, gvec[k], gsize)

        bufs = [buf0, buf1]
        in_sems = [in_s0, in_s1]
        out_sems = [out_s0, out_s1]

        din = [pltpu.async_copy(V_hbm.at[bidx, pl.ds(r0 + c * _C, _C)],
                                bufs[c].at[:, pl.ds(0, _OV)], in_sems[c])
               for c in range(2)]

        gv = [gs_row_v[pl.ds(j * 16, 16)] for j in range(_NV)]
        zv = [jnp.zeros((16,), jnp.float32)] * _NV
        _fill_tail(buf0, gv)
        _fill_tail(buf1, gv)

        dout = [None, None]
        for c in range(_NC):
            s = c & 1
            base = r0 + c * _C
            if c >= 2:
                dout[s].wait()
                din[s] = pltpu.async_copy(
                    V_hbm.at[bidx, pl.ds(base, _C)],
                    bufs[s].at[:, pl.ds(0, _OV)], in_sems[s])
            din[s].wait()

            is_zero = base >= gsize
            is_mixed = jnp.logical_and(base < gsize, base + _C > gsize)
            if c >= 2:
                prev_zero = (base - 2 * _C) >= gsize
                need_zero = jnp.logical_and(is_zero,
                                            jnp.logical_not(prev_zero))
            else:
                need_zero = is_zero

            @pl.when(need_zero)
            def _():
                _fill_tail(bufs[s], zv)

            @pl.when(is_mixed)
            def _():
                def body(i, _):
                    m = jnp.where(base + i < gsize, 1.0, 0.0)
                    for j in range(_NV):
                        bufs[s][i, pl.ds(_OV + j * 16, 16)] = gv[j] * m
                    return 0
                lax.fori_loop(0, _C, body, 0)

            dout[s] = pltpu.async_copy(bufs[s],
                                       out_hbm.at[lb, pl.ds(base, _C)],
                                       out_sems[s])

        dout[0].wait()
        dout[1].wait()
    return _sc_body


@jax.jit
def kernel(V, global_state, graph_size, W):
    b, N, Ov = V.shape
    O = W.shape[1]
    gs = pl.pallas_call(
        _gs_body,
        out_shape=jax.ShapeDtypeStruct((b, O), jnp.float32),
    )(global_state, W)

    parts = []
    for p in range(_PARTS):
        sc_assemble = pl.kernel(
            _make_sc_body(p),
            out_type=jax.ShapeDtypeStruct((_BPP, N, Ov + O), jnp.float32),
            mesh=plsc.VectorSubcoreMesh(core_axis_name="c",
                                        subcore_axis_name="s"),
            compiler_params=pltpu.CompilerParams(use_tc_tiling_on_sc=True),
            scratch_types=[
                pltpu.VMEM((b,), jnp.int32),
                pltpu.VMEM((O,), jnp.float32),
                pltpu.VMEM((_C, Ov + O), jnp.float32),
                pltpu.VMEM((_C, Ov + O), jnp.float32),
                pltpu.SemaphoreType.DMA,
                pltpu.SemaphoreType.DMA,
                pltpu.SemaphoreType.DMA,
                pltpu.SemaphoreType.DMA,
            ],
            name=f"sc_assemble_p{p}",
        )
        parts.append(sc_assemble(V, gs, graph_size))
    out = jnp.concatenate(parts, axis=0)
    return out, gs


# speedup vs baseline: 1.6414x; 1.6414x over previous
import functools

import jax
import jax.numpy as jnp
from jax import lax
from jax.experimental import pallas as pl
from jax.experimental.pallas import tpu as pltpu
from jax.experimental.pallas import tpu_sc as plsc

_B, _N, _OV, _O = 16, 4096, 128, 64
_NW = 32
_RW = _B * _N // _NW
_C = 256
_NC = _RW // _C
_NV = _O // 16


def _gs_body(global_state_ref, W_ref, gs_ref):
    gs_ref[...] = jnp.dot(global_state_ref[...], W_ref[...],
                          preferred_element_type=jnp.float32)


def _rep_body(gs_hbm, gsz_hbm, rep_hbm, gsz_v, gs_row_v, tmpl, sem):
    cid = lax.axis_index("c")
    sid = lax.axis_index("s")
    wid = sid * 2 + cid
    bidx = wid // 2
    r0 = (wid % 2) * _RW

    pltpu.sync_copy(gsz_hbm, gsz_v)
    pltpu.sync_copy(gs_hbm.at[bidx], gs_row_v)
    gvec = gsz_v[...]
    gsize = gvec[0]
    for k in range(1, _B):
        gsize = jnp.where(bidx == k, gvec[k], gsize)

    gv = [gs_row_v[pl.ds(j * 16, 16)] for j in range(_NV)]
    zv = [jnp.zeros((16,), jnp.float32)] * _NV

    def fill(i, _):
        for j in range(_NV):
            tmpl[i, pl.ds(j * 16, 16)] = gv[j]
            tmpl[_C + i, pl.ds(j * 16, 16)] = zv[j]
        return 0
    lax.fori_loop(0, _C, fill, 0)

    descs = []
    for c in range(_NC):
        base = r0 + c * _C
        off = jnp.clip(_C - (gsize - base), 0, _C)
        descs.append(pltpu.async_copy(
            tmpl.at[pl.ds(off, _C)],
            rep_hbm.at[bidx, pl.ds(base, _C), pl.ds(0, _O)], sem))
    for d in descs:
        d.wait()


@jax.jit
def kernel(V, global_state, graph_size, W):
    b, N, Ov = V.shape
    O = W.shape[1]
    gs = pl.pallas_call(
        _gs_body,
        out_shape=jax.ShapeDtypeStruct((b, O), jnp.float32),
    )(global_state, W)

    rep_build = pl.kernel(
        _rep_body,
        out_type=jax.ShapeDtypeStruct((b, N, 2 * O), jnp.float32),
        mesh=plsc.VectorSubcoreMesh(core_axis_name="c", subcore_axis_name="s"),
        compiler_params=pltpu.CompilerParams(use_tc_tiling_on_sc=False),
        scratch_types=[
            pltpu.VMEM((b,), jnp.int32),
            pltpu.VMEM((O,), jnp.float32),
            pltpu.VMEM((2 * _C, O), jnp.float32),
            pltpu.SemaphoreType.DMA,
        ],
        name="sc_rep_build",
    )
    rep = rep_build(gs, graph_size)
    out = jnp.concatenate([V, rep[:, :, :O]], axis=-1)
    return out, gs
